# baseline (speedup 1.0000x reference)
import functools

import jax
import jax.numpy as jnp
from jax import lax
from jax.experimental import pallas as pl
from jax.experimental.pallas import tpu as pltpu
from jax.experimental.pallas import tpu_sc as plsc

VOCAB = 1000000
EMBED_DIM = 64
BATCH = 4096
HIST = 200

NW = 32
TOTAL = BATCH * HIST
PER_W = TOTAL // NW
CHUNK = 128
NCHUNKS = PER_W // CHUNK
NB = 8
NGROUPS = NCHUNKS // NB


def _body(idx_hbm, table_hbm, out_hbm, idx_v, rows, gsems, csems):
    wid = lax.axis_index("s") * 2 + lax.axis_index("c")
    out_base = wid * PER_W

    pltpu.sync_copy(idx_hbm.at[wid], idx_v)

    def gather_start(chunk, slot):
        pltpu.async_copy(table_hbm.at[idx_v.at[chunk]], rows[slot], gsems[slot])

    def gather_wait(slot):
        pltpu.make_async_copy(table_hbm.at[idx_v.at[0]], rows[slot],
                              gsems[slot]).wait()

    def copyout_start(chunk, slot):
        pltpu.async_copy(rows[slot],
                         out_hbm.at[pl.ds(out_base + chunk * CHUNK, CHUNK)],
                         csems[slot])

    def copyout_wait(slot):
        pltpu.make_async_copy(rows[slot], out_hbm.at[pl.ds(0, CHUNK)],
                              csems[slot]).wait()

    for b in range(NB):
        gather_start(b, b)

    @pl.loop(0, NGROUPS)
    def group(g):
        base = g * NB
        for b in range(NB):
            gather_wait(b)
            copyout_start(base + b, b)
        for b in range(NB):
            copyout_wait(b)

            @pl.when(g < NGROUPS - 1)
            def _():
                gather_start(base + NB + b, b)


@functools.partial(
    pl.kernel,
    out_type=jax.ShapeDtypeStruct((TOTAL, EMBED_DIM), jnp.float32),
    mesh=plsc.VectorSubcoreMesh(core_axis_name="c", subcore_axis_name="s"),
    scratch_types=(
        [pltpu.VMEM((NCHUNKS, CHUNK), jnp.int32)]
        + [pltpu.VMEM((CHUNK, EMBED_DIM), jnp.float32) for _ in range(NB)]
        + [pltpu.SemaphoreType.DMA for _ in range(2 * NB)]
    ),
    compiler_params=pltpu.CompilerParams(use_tc_tiling_on_sc=False),
)
def _gather_kernel(idx_hbm, table_hbm, out_hbm, idx_v, *bufs):
    rows = bufs[:NB]
    gsems = bufs[NB:2 * NB]
    csems = bufs[2 * NB:]
    _body(idx_hbm, table_hbm, out_hbm, idx_v, rows, gsems, csems)


@jax.jit
def kernel(indices, table):
    idx = indices.astype(jnp.int32).reshape(NW, NCHUNKS, CHUNK)
    out = _gather_kernel(idx, table)
    return out.reshape(BATCH, HIST, EMBED_DIM)
